# baseline (speedup 1.0000x reference)
import functools

import jax
import jax.numpy as jnp
from jax import lax
from jax.experimental import pallas as pl
from jax.experimental.pallas import tpu as pltpu
from jax.experimental.pallas import tpu_sc as plsc

_NUM_CORES = 2
_NUM_SUBCORES = 16
_NUM_TILES = _NUM_CORES * _NUM_SUBCORES
_LANES = 128
_GBUF = 4
_TBUF = 2


def kernel(word, table):
    bsz, seq = word.shape
    num_idx = bsz * seq
    dim = table.shape[1]
    vocab = table.shape[0]
    n_per_tile = num_idx // _NUM_TILES
    assert bsz == _LANES * _NUM_TILES and n_per_tile == _LANES * seq
    assert dim == 64 and seq >= 4 and (seq - 4) % _GBUF == 0

    idx = word.reshape(num_idx).astype(jnp.int32)
    tbl2 = jnp.pad(table, ((0, 0), (0, dim))).reshape(2 * vocab, dim)
    mesh = plsc.VectorSubcoreMesh(core_axis_name="c", subcore_axis_name="s")

    @functools.partial(
        pl.kernel,
        out_type=jax.ShapeDtypeStruct((seq, 8, _NUM_TILES, 8, _LANES), table.dtype),
        mesh=mesh,
        scratch_types=[
            pltpu.VMEM((n_per_tile,), jnp.int32),
            pltpu.VMEM((n_per_tile,), jnp.int32),
            pltpu.VMEM((_GBUF, _LANES, dim), table.dtype),
            pltpu.VMEM((_TBUF, dim, _LANES), table.dtype),
            pltpu.SemaphoreType.DMA((_GBUF,)),
            pltpu.SemaphoreType.DMA((_TBUF,)),
        ],
        compiler_params=pltpu.CompilerParams(
            use_tc_tiling_on_sc=False, needs_layout_passes=False
        ),
    )
    def gather_kernel(tbl_hbm, idx_hbm, out_hbm, idx_v, idxp_v, g_v, t_v, g_sem, w_sem):
        wid = lax.axis_index("s") * _NUM_CORES + lax.axis_index("c")
        base = wid * n_per_tile
        pltpu.sync_copy(idx_hbm.at[pl.ds(base, n_per_tile)], idx_v)

        iota16 = lax.iota(jnp.int32, 16)

        rvecs = [iota16 + 16 * g for g in range(_LANES // 16)]

        @pl.loop(0, seq)
        def _(s):
            for g in range(_LANES // 16):
                src = rvecs[g] * seq + s
                v = plsc.load_gather(idx_v, [src])
                idxp_v[pl.ds(s * _LANES + 16 * g, 16)] = v * 2

        def gather_copy(c, b):
            return pltpu.make_async_copy(
                tbl_hbm.at[idxp_v.at[pl.ds(c * _LANES, _LANES)]],
                g_v.at[b],
                g_sem.at[b],
            )

        def transpose(c, b, tb):
            @pl.loop(0, dim // 4)
            def _(q):
                for h in range(4):
                    d = q * 4 + h
                    dvec = iota16 * 0 + d
                    for g in range(_LANES // 16):
                        vals = plsc.load_gather(g_v.at[b], [rvecs[g], dvec])
                        t_v[tb, d, pl.ds(16 * g, 16)] = vals

        def write_copies(c, tb):
            return [
                pltpu.make_async_copy(
                    t_v.at[tb, pl.ds(8 * k, 8)],
                    out_hbm.at[c, k, wid],
                    w_sem.at[tb],
                )
                for k in range(8)
            ]

        def start_writes(c, tb):
            for cp in write_copies(c, tb):
                cp.start()

        def wait_writes(c, tb):
            for cp in write_copies(c, tb):
                cp.wait()

        def step(c, b, tb, head=False):
            gather_copy(c, b).wait()
            if not head:
                wait_writes(c - _TBUF, tb)
            transpose(c, b, tb)
            start_writes(c, tb)

        gather_copy(0, 0).start()
        gather_copy(1, 1).start()
        for c in range(2):
            gather_copy(c + 2, c + 2).start()
            step(c, c, c, head=True)

        @pl.loop(0, (seq - 4) // _GBUF)
        def _(p):
            for k in range(_GBUF):
                c = 2 + p * _GBUF + k
                b = (2 + k) % _GBUF
                gather_copy(c + 2, (b + 2) % _GBUF).start()
                step(c, b, k % _TBUF)

        for i in range(2):
            c = seq - 2 + i
            step(c, c % _GBUF, c % _TBUF)
        for i in range(2):
            wait_writes(seq - 2 + i, (seq - 2 + i) % _TBUF)

    out5 = gather_kernel(tbl2, idx)
    return jnp.transpose(out5, (2, 4, 0, 1, 3)).reshape(bsz, seq, dim)


# speedup vs baseline: 1.6629x; 1.6629x over previous
import functools

import jax
import jax.numpy as jnp
from jax import lax
from jax.experimental import pallas as pl
from jax.experimental.pallas import tpu as pltpu
from jax.experimental.pallas import tpu_sc as plsc

_NUM_CORES = 2
_NUM_SUBCORES = 16
_NUM_TILES = _NUM_CORES * _NUM_SUBCORES
_LANES = 128
_GBUF = 4
_TBUF = 2


def kernel(word, table):
    bsz, seq = word.shape
    num_idx = bsz * seq
    dim = table.shape[1]
    vocab = table.shape[0]
    n_per_tile = num_idx // _NUM_TILES
    assert bsz == _LANES * _NUM_TILES and n_per_tile == _LANES * seq
    assert dim == 64 and seq >= 4 and (seq - 4) % _GBUF == 0

    idx = word.reshape(num_idx).astype(jnp.int32)
    tbl2 = jnp.pad(table, ((0, 0), (0, dim))).reshape(2 * vocab, dim)
    mesh = plsc.VectorSubcoreMesh(core_axis_name="c", subcore_axis_name="s")

    @functools.partial(
        pl.kernel,
        out_type=jax.ShapeDtypeStruct((seq, 8, _NUM_TILES, 8 * _LANES), table.dtype),
        mesh=mesh,
        scratch_types=[
            pltpu.VMEM((n_per_tile,), jnp.int32),
            pltpu.VMEM((n_per_tile,), jnp.int32),
            pltpu.VMEM((_GBUF, _LANES, dim), table.dtype),
            pltpu.VMEM((_TBUF, dim * _LANES), table.dtype),
            pltpu.SemaphoreType.DMA((_GBUF,)),
            pltpu.SemaphoreType.DMA((_TBUF,)),
        ],
        compiler_params=pltpu.CompilerParams(
            use_tc_tiling_on_sc=False, needs_layout_passes=False
        ),
    )
    def gather_kernel(tbl_hbm, idx_hbm, out_hbm, idx_v, idxp_v, g_v, t_v, g_sem, w_sem):
        wid = lax.axis_index("s") * _NUM_CORES + lax.axis_index("c")
        base = wid * n_per_tile
        pltpu.sync_copy(idx_hbm.at[pl.ds(base, n_per_tile)], idx_v)

        iota16 = lax.iota(jnp.int32, 16)

        rvecs = [iota16 + 16 * g for g in range(_LANES // 16)]

        @pl.loop(0, seq)
        def _(s):
            for g in range(_LANES // 16):
                src = rvecs[g] * seq + s
                v = plsc.load_gather(idx_v, [src])
                idxp_v[pl.ds(s * _LANES + 16 * g, 16)] = v * 2

        def gather_copy(c, b):
            return pltpu.make_async_copy(
                tbl_hbm.at[idxp_v.at[pl.ds(c * _LANES, _LANES)]],
                g_v.at[b],
                g_sem.at[b],
            )

        rot = [jnp.bitwise_and(iota16 + dd, 15) for dd in range(16)]
        srot = [rot[dd] * _LANES + iota16 for dd in range(16)]

        def transpose(c, b, tb):
            @pl.loop(0, _LANES // 16)
            def _(gq):
                bb0 = gq * 16
                rowvec = iota16 + bb0
                for d0 in range(0, dim, 16):
                    base = d0 * _LANES + bb0
                    for dd in range(16):
                        colvec = rot[dd] + d0
                        vals = plsc.load_gather(g_v.at[b], [rowvec, colvec])
                        svec = srot[dd] + base
                        plsc.store_scatter(t_v.at[tb], [svec], vals)

        def write_copies(c, tb):
            return [
                pltpu.make_async_copy(
                    t_v.at[tb, pl.ds(1024 * k, 1024)],
                    out_hbm.at[c, k, wid],
                    w_sem.at[tb],
                )
                for k in range(8)
            ]

        def start_writes(c, tb):
            for cp in write_copies(c, tb):
                cp.start()

        def wait_writes(c, tb):
            for cp in write_copies(c, tb):
                cp.wait()

        def step(c, b, tb, head=False):
            gather_copy(c, b).wait()
            if not head:
                wait_writes(c - _TBUF, tb)
            transpose(c, b, tb)
            start_writes(c, tb)

        gather_copy(0, 0).start()
        gather_copy(1, 1).start()
        for c in range(2):
            gather_copy(c + 2, c + 2).start()
            step(c, c, c, head=True)

        @pl.loop(0, (seq - 4) // _GBUF)
        def _(p):
            for k in range(_GBUF):
                c = 2 + p * _GBUF + k
                b = (2 + k) % _GBUF
                gather_copy(c + 2, (b + 2) % _GBUF).start()
                step(c, b, k % _TBUF)

        for i in range(2):
            c = seq - 2 + i
            step(c, c % _GBUF, c % _TBUF)
        for i in range(2):
            wait_writes(seq - 2 + i, (seq - 2 + i) % _TBUF)

    out5 = gather_kernel(tbl2, idx)
    out5 = out5.reshape(seq, 8, _NUM_TILES, 8, _LANES)
    return jnp.transpose(out5, (2, 4, 0, 1, 3)).reshape(bsz, seq, dim)


# speedup vs baseline: 2.1841x; 1.3134x over previous
import functools

import jax
import jax.numpy as jnp
from jax import lax
from jax.experimental import pallas as pl
from jax.experimental.pallas import tpu as pltpu
from jax.experimental.pallas import tpu_sc as plsc

_NUM_CORES = 2
_NUM_SUBCORES = 16
_NUM_TILES = _NUM_CORES * _NUM_SUBCORES
_LANES = 128
_GBUF = 4
_TBUF = 2


def kernel(word, table):
    bsz, seq = word.shape
    num_idx = bsz * seq
    dim = table.shape[1]
    vocab = table.shape[0]
    n_per_tile = num_idx // _NUM_TILES
    assert bsz == _LANES * _NUM_TILES and n_per_tile == _LANES * seq
    assert dim == 64 and seq >= 4 and (seq - 4) % _GBUF == 0

    idx = word.reshape(num_idx).astype(jnp.int32)
    tbl2 = jnp.pad(table, ((0, 0), (0, dim))).reshape(2 * vocab, dim)
    mesh = plsc.VectorSubcoreMesh(core_axis_name="c", subcore_axis_name="s")

    @functools.partial(
        pl.kernel,
        out_type=jax.ShapeDtypeStruct((seq, 8, _NUM_TILES, 8 * _LANES), table.dtype),
        mesh=mesh,
        scratch_types=[
            pltpu.VMEM((n_per_tile,), jnp.int32),
            pltpu.VMEM((n_per_tile,), jnp.int32),
            pltpu.VMEM((_GBUF, _LANES, dim), table.dtype),
            pltpu.VMEM((_TBUF, dim * _LANES), table.dtype),
            pltpu.SemaphoreType.DMA((_GBUF,)),
            pltpu.SemaphoreType.DMA((_TBUF,)),
        ],
        compiler_params=pltpu.CompilerParams(
            use_tc_tiling_on_sc=False, needs_layout_passes=False
        ),
    )
    def gather_kernel(tbl_hbm, idx_hbm, out_hbm, idx_v, idxp_v, g_v, t_v, g_sem, w_sem):
        wid = lax.axis_index("s") * _NUM_CORES + lax.axis_index("c")
        base = wid * n_per_tile
        pltpu.sync_copy(idx_hbm.at[pl.ds(base, n_per_tile)], idx_v)

        iota16 = lax.iota(jnp.int32, 16)

        rvecs = [iota16 + 16 * g for g in range(_LANES // 16)]

        @pl.loop(0, seq)
        def _(s):
            for g in range(_LANES // 16):
                src = rvecs[g] * seq + s
                v = plsc.load_gather(idx_v, [src])
                idxp_v[pl.ds(s * _LANES + 16 * g, 16)] = v * 2

        def gather_copy(c, b):
            return pltpu.make_async_copy(
                tbl_hbm.at[idxp_v.at[pl.ds(c * _LANES, _LANES)]],
                g_v.at[b],
                g_sem.at[b],
            )

        rot = [jnp.bitwise_and(iota16 + dd, 15) for dd in range(16)]
        srot = [rot[dd] * _LANES + iota16 for dd in range(16)]

        def transpose(c, b, tb):
            @pl.loop(0, _LANES // 16)
            def _(gq):
                bb0 = gq * 16
                rowvec = iota16 + bb0
                for d0 in range(0, dim, 16):
                    base = d0 * _LANES + bb0
                    for dd0 in range(0, 16, 4):
                        vals4 = [
                            plsc.load_gather(g_v.at[b], [rowvec, rot[dd0 + i] + d0])
                            for i in range(4)
                        ]
                        for i in range(4):
                            plsc.store_scatter(
                                t_v.at[tb], [srot[dd0 + i] + base], vals4[i]
                            )

        def write_copies(c, tb):
            return [
                pltpu.make_async_copy(
                    t_v.at[tb, pl.ds(1024 * k, 1024)],
                    out_hbm.at[c, k, wid],
                    w_sem.at[tb],
                )
                for k in range(8)
            ]

        def start_writes(c, tb):
            for cp in write_copies(c, tb):
                cp.start()

        def wait_writes(c, tb):
            for cp in write_copies(c, tb):
                cp.wait()

        def step(c, b, tb, head=False):
            gather_copy(c, b).wait()
            if not head:
                wait_writes(c - _TBUF, tb)
            transpose(c, b, tb)
            start_writes(c, tb)

        gather_copy(0, 0).start()
        gather_copy(1, 1).start()
        for c in range(2):
            gather_copy(c + 2, c + 2).start()
            step(c, c, c, head=True)

        @pl.loop(0, (seq - 4) // _GBUF)
        def _(p):
            for k in range(_GBUF):
                c = 2 + p * _GBUF + k
                b = (2 + k) % _GBUF
                gather_copy(c + 2, (b + 2) % _GBUF).start()
                step(c, b, k % _TBUF)

        for i in range(2):
            c = seq - 2 + i
            step(c, c % _GBUF, c % _TBUF)
        for i in range(2):
            wait_writes(seq - 2 + i, (seq - 2 + i) % _TBUF)

    out5 = gather_kernel(tbl2, idx)
    out5 = out5.reshape(seq, 8, _NUM_TILES, 8, _LANES)
    return jnp.transpose(out5, (2, 4, 0, 1, 3)).reshape(bsz, seq, dim)


# speedup vs baseline: 2.5211x; 1.1543x over previous
import functools

import jax
import jax.numpy as jnp
from jax import lax
from jax.experimental import pallas as pl
from jax.experimental.pallas import tpu as pltpu
from jax.experimental.pallas import tpu_sc as plsc

_NUM_CORES = 2
_NUM_SUBCORES = 16
_NUM_TILES = _NUM_CORES * _NUM_SUBCORES
_LANES = 128
_GBUF = 4
_TBUF = 2


def kernel(word, table):
    bsz, seq = word.shape
    num_idx = bsz * seq
    dim = table.shape[1]
    vocab = table.shape[0]
    n_per_tile = num_idx // _NUM_TILES
    assert bsz == _LANES * _NUM_TILES and n_per_tile == _LANES * seq
    assert dim == 64 and seq >= 4 and (seq - 4) % _GBUF == 0

    idx = word.reshape(num_idx).astype(jnp.int32)
    tbl2 = jnp.pad(table, ((0, 0), (0, dim))).reshape(2 * vocab, dim)
    mesh = plsc.VectorSubcoreMesh(core_axis_name="c", subcore_axis_name="s")

    @functools.partial(
        pl.kernel,
        out_type=jax.ShapeDtypeStruct((seq, 8, _NUM_TILES, 8 * _LANES), table.dtype),
        mesh=mesh,
        scratch_types=[
            pltpu.VMEM((n_per_tile,), jnp.int32),
            pltpu.VMEM((n_per_tile,), jnp.int32),
            pltpu.VMEM((_GBUF, _LANES, dim), table.dtype),
            pltpu.VMEM((_TBUF, dim * _LANES), table.dtype),
            pltpu.SemaphoreType.DMA((_GBUF,)),
            pltpu.SemaphoreType.DMA((_TBUF,)),
        ],
        compiler_params=pltpu.CompilerParams(
            use_tc_tiling_on_sc=False, needs_layout_passes=False
        ),
    )
    def gather_kernel(tbl_hbm, idx_hbm, out_hbm, idx_v, idxp_v, g_v, t_v, g_sem, w_sem):
        wid = lax.axis_index("s") * _NUM_CORES + lax.axis_index("c")
        base = wid * n_per_tile
        pltpu.sync_copy(idx_hbm.at[pl.ds(base, n_per_tile)], idx_v)

        iota16 = lax.iota(jnp.int32, 16)

        rvecs = [iota16 + 16 * g for g in range(_LANES // 16)]

        @pl.loop(0, seq)
        def _(s):
            for g in range(_LANES // 16):
                src = rvecs[g] * seq + s
                v = plsc.load_gather(idx_v, [src])
                idxp_v[pl.ds(s * _LANES + 16 * g, 16)] = v * 2

        def gather_copy(c, b):
            return pltpu.make_async_copy(
                tbl_hbm.at[idxp_v.at[pl.ds(c * _LANES, _LANES)]],
                g_v.at[b],
                g_sem.at[b],
            )

        rot = [jnp.bitwise_and(iota16 + dd, 15) for dd in range(16)]
        srot = [rot[dd] * _LANES + iota16 for dd in range(16)]

        def transpose(c, b, tb):
            @pl.loop(0, _LANES // 16)
            def _(gq):
                bb0 = gq * 16
                rowvec = iota16 + bb0
                for d0 in range(0, dim, 16):
                    base = d0 * _LANES + bb0
                    for dd0 in range(0, 16, 8):
                        vals8 = [
                            plsc.load_gather(g_v.at[b], [rowvec, rot[dd0 + i] + d0])
                            for i in range(8)
                        ]
                        for i in range(8):
                            plsc.store_scatter(
                                t_v.at[tb], [srot[dd0 + i] + base], vals8[i]
                            )

        def write_copies(c, tb):
            return [
                pltpu.make_async_copy(
                    t_v.at[tb, pl.ds(1024 * k, 1024)],
                    out_hbm.at[c, k, wid],
                    w_sem.at[tb],
                )
                for k in range(8)
            ]

        def start_writes(c, tb):
            for cp in write_copies(c, tb):
                cp.start()

        def wait_writes(c, tb):
            for cp in write_copies(c, tb):
                cp.wait()

        def step(c, b, tb, head=False):
            gather_copy(c, b).wait()
            if not head:
                wait_writes(c - _TBUF, tb)
            transpose(c, b, tb)
            start_writes(c, tb)

        gather_copy(0, 0).start()
        gather_copy(1, 1).start()
        for c in range(2):
            gather_copy(c + 2, c + 2).start()
            step(c, c, c, head=True)

        @pl.loop(0, (seq - 4) // _GBUF)
        def _(p):
            for k in range(_GBUF):
                c = 2 + p * _GBUF + k
                b = (2 + k) % _GBUF
                gather_copy(c + 2, (b + 2) % _GBUF).start()
                step(c, b, k % _TBUF)

        for i in range(2):
            c = seq - 2 + i
            step(c, c % _GBUF, c % _TBUF)
        for i in range(2):
            wait_writes(seq - 2 + i, (seq - 2 + i) % _TBUF)

    out5 = gather_kernel(tbl2, idx)
    out5 = out5.reshape(seq, 8, _NUM_TILES, 8, _LANES)
    return jnp.transpose(out5, (2, 4, 0, 1, 3)).reshape(bsz, seq, dim)
